# baseline (speedup 1.0000x reference)
import functools

import jax
import jax.numpy as jnp
from jax import lax
from jax.experimental import pallas as pl
from jax.experimental.pallas import tpu as pltpu
from jax.experimental.pallas import tpu_sc as plsc

NC, NS = 2, 16
NW = NC * NS
S, T = 4, 4096
D = 1024
BPW = (S * T) // NW
WPS = T // BPW
CHUNK = 32
NCHUNK = BPW // CHUNK
ROUNDS = NCHUNK // 2

_mesh = plsc.VectorSubcoreMesh(
    core_axis_name="c", subcore_axis_name="s", num_cores=NC, num_subcores=NS
)


@functools.partial(
    pl.kernel,
    out_type=jax.ShapeDtypeStruct((S, T, D), jnp.float32),
    mesh=_mesh,
    scratch_types=[
        pltpu.VMEM((2, CHUNK), jnp.int32),
        pltpu.VMEM((2, CHUNK, D), jnp.float32),
        pltpu.SemaphoreType.DMA,
        pltpu.SemaphoreType.DMA,
        pltpu.SemaphoreType.DMA,
        pltpu.SemaphoreType.DMA,
        pltpu.SemaphoreType.DMA,
        pltpu.SemaphoreType.DMA,
    ],
)
def _embed(tokens_hbm, table_hbm, out_hbm, idx_v, rows_v, i0, i1, g0, g1, s0, s1):
    isems = (i0, i1)
    gsems = (g0, g1)
    ssems = (s0, s1)
    wid = lax.axis_index("s") * NC + lax.axis_index("c")
    seq = wid // WPS
    col0 = (wid % WPS) * BPW

    def icopy(c, j):
        return pltpu.make_async_copy(
            tokens_hbm.at[seq, pl.ds(col0 + c * CHUNK, CHUNK)], idx_v.at[j], isems[j]
        )

    def gather(j):
        return pltpu.make_async_copy(
            table_hbm.at[idx_v.at[j]], rows_v.at[j], gsems[j]
        )

    def store(c, j):
        return pltpu.make_async_copy(
            rows_v.at[j], out_hbm.at[seq, pl.ds(col0 + c * CHUNK, CHUNK)], ssems[j]
        )

    icopy(0, 0).start()
    icopy(1, 1).start()
    for j in range(2):
        icopy(j, j).wait()
        gather(j).start()

    @pl.loop(0, ROUNDS - 1)
    def _round(r):
        c0 = 2 * r
        for j in range(2):
            gather(j).wait()
            icopy(c0 + j + 2, j).start()
            store(c0 + j, j).start()
        for j in range(2):
            store(c0 + j, j).wait()
            icopy(c0 + j + 2, j).wait()
            gather(j).start()

    c0 = 2 * (ROUNDS - 1)
    for j in range(2):
        gather(j).wait()
        store(c0 + j, j).start()
    for j in range(2):
        store(c0 + j, j).wait()


def kernel(tokens, W_E):
    return _embed(tokens, W_E)


# speedup vs baseline: 1.0511x; 1.0511x over previous
import functools

import jax
import jax.numpy as jnp
from jax import lax
from jax.experimental import pallas as pl
from jax.experimental.pallas import tpu as pltpu
from jax.experimental.pallas import tpu_sc as plsc

NC, NS = 2, 16
NW = NC * NS
S, T = 4, 4096
D = 1024
BPW = (S * T) // NW
WPS = T // BPW
CHUNK = 56
NFULL = BPW // CHUNK
TAIL = BPW - NFULL * CHUNK
SIZES = [CHUNK] * NFULL + ([TAIL] if TAIL else [])
OFFS = [i * CHUNK for i in range(NFULL)] + ([NFULL * CHUNK] if TAIL else [])

_mesh = plsc.VectorSubcoreMesh(
    core_axis_name="c", subcore_axis_name="s", num_cores=NC, num_subcores=NS
)


@functools.partial(
    pl.kernel,
    out_type=jax.ShapeDtypeStruct((S, T, D), jnp.float32),
    mesh=_mesh,
    scratch_types=[
        pltpu.VMEM((2, CHUNK), jnp.int32),
        pltpu.VMEM((max(TAIL,8),), jnp.int32),
        pltpu.VMEM((2, CHUNK, D), jnp.float32),
        pltpu.VMEM((max(TAIL,8), D), jnp.float32),
        pltpu.SemaphoreType.DMA,
        pltpu.SemaphoreType.DMA,
        pltpu.SemaphoreType.DMA,
        pltpu.SemaphoreType.DMA,
        pltpu.SemaphoreType.DMA,
        pltpu.SemaphoreType.DMA,
        pltpu.SemaphoreType.DMA,
    ],
)
def _embed(tokens_hbm, table_hbm, out_hbm, idx_v, idxt_v, rows_v, rowst_v,
           i0, i1, it, g0, g1, s0, s1):
    isems = (i0, i1)
    gsems = (g0, g1)
    ssems = (s0, s1)
    wid = lax.axis_index("s") * NC + lax.axis_index("c")
    seq = wid // WPS
    col0 = (wid % WPS) * BPW
    nch = len(SIZES)

    def ids_src(c):
        return tokens_hbm.at[pl.ds(seq * T + col0 + OFFS[c], SIZES[c])]

    def idx_ref(c, j):
        return idxt_v if (TAIL and c == nch - 1) else idx_v.at[j]

    def buf(c, j):
        if SIZES[c] == CHUNK:
            return rows_v.at[j]
        return rowst_v.at[pl.ds(0, SIZES[c])]

    def icopy(c, j):
        sem = it if (TAIL and c == nch - 1) else isems[j]
        return pltpu.make_async_copy(ids_src(c), idx_ref(c, j), sem)

    def gather(c, j):
        return pltpu.make_async_copy(
            table_hbm.at[idx_ref(c, j)], buf(c, j), gsems[j]
        )

    def store(c, j):
        return pltpu.make_async_copy(
            buf(c, j), out_hbm.at[seq, pl.ds(col0 + OFFS[c], SIZES[c])], ssems[j]
        )

    icopy(0, 0).start()
    icopy(1, 1).start()
    if TAIL:
        icopy(nch - 1, (nch - 1) % 2).start()
    for j in range(2):
        icopy(j, j).wait()
        gather(j, j).start()

    for c in range(nch):
        j = c % 2
        nc = c + 2
        gather(c, j).wait()
        if nc < nch - 1:
            icopy(nc, j).start()
        store(c, j).start()
        if nc < nch:
            store(c, j).wait()
            icopy(nc, j).wait()
            gather(nc, j).start()

    for c in range(nch - 2, nch):
        store(c, c % 2).wait()


def kernel(tokens, W_E):
    return _embed(tokens.reshape(-1), W_E)
